# Initial kernel scaffold; baseline (speedup 1.0000x reference)
import functools

import jax
import jax.numpy as jnp
from jax import lax
from jax.experimental import pallas as pl
from jax.experimental.pallas import tpu as pltpu
from jax.experimental.pallas import tpu_sc as plsc

N = 10000
E = 320000
D_IN = 128
D_HID = 128
D_OUT = 64

NC = 2
NS = 16
NT = NC * NS
NPAD = 10240
SLAB = NPAD // NS
CH = 80
NCH = (E // NT) // CH

_MESH = plsc.VectorSubcoreMesh(
    core_axis_name="c", subcore_axis_name="s", num_cores=NC, num_subcores=NS
)


@functools.partial(
    pl.kernel,
    out_type=jax.ShapeDtypeStruct((NC * NPAD,), jnp.float32),
    mesh=_MESH,
    scratch_types=[
        pltpu.VMEM((NCH, CH), jnp.int32),
        pltpu.VMEM((CH,), jnp.float32),
        pltpu.VMEM_SHARED((NPAD,), jnp.float32),
    ],
)
def _deg_kernel(dst2d, zeros1, deg_out, idx_d, ones_v, sdeg):
    c = lax.axis_index("c")
    s = lax.axis_index("s")
    t = c * NS + s
    pltpu.sync_copy(zeros1.at[pl.ds(s * SLAB, SLAB)], sdeg.at[pl.ds(s * SLAB, SLAB)])
    pltpu.sync_copy(dst2d.at[pl.ds(t * NCH, NCH)], idx_d)
    ones16 = jnp.full((16,), 1.0, dtype=jnp.float32)
    for j in range(CH // 16):
        ones_v[pl.ds(j * 16, 16)] = ones16
    plsc.subcore_barrier()

    def body(i, carry):
        pltpu.sync_copy(ones_v, sdeg.at[idx_d.at[i]], add=True)
        return carry

    lax.fori_loop(0, NCH, body, 0)
    plsc.subcore_barrier()
    pltpu.sync_copy(
        sdeg.at[pl.ds(s * SLAB, SLAB)],
        deg_out.at[pl.ds(c * NPAD + s * SLAB, SLAB)],
    )


def _make_scatter(D):
    @functools.partial(
        pl.kernel,
        out_type=jax.ShapeDtypeStruct((NC * NPAD, D), jnp.float32),
        mesh=_MESH,
        scratch_types=[
            pltpu.VMEM((NCH, CH), jnp.int32),
            pltpu.VMEM((NCH, CH), jnp.int32),
            pltpu.VMEM((2, CH, D), jnp.float32),
            pltpu.SemaphoreType.DMA,
            pltpu.VMEM_SHARED((NPAD, D), jnp.float32),
        ],
    )
    def _scatter(h, src2d, dst2d, zeros2, acc_out, idx_s, idx_d, rows, sem, sacc):
        c = lax.axis_index("c")
        s = lax.axis_index("s")
        t = c * NS + s
        pltpu.sync_copy(
            zeros2.at[pl.ds(s * SLAB, SLAB)], sacc.at[pl.ds(s * SLAB, SLAB)]
        )
        pltpu.sync_copy(src2d.at[pl.ds(t * NCH, NCH)], idx_s)
        pltpu.sync_copy(dst2d.at[pl.ds(t * NCH, NCH)], idx_d)
        plsc.subcore_barrier()
        pltpu.async_copy(h.at[idx_s.at[0]], rows.at[0], sem)

        def body(i, carry):
            p = lax.rem(i, 2)
            pn = lax.rem(i + 1, 2)
            pltpu.make_async_copy(h.at[idx_s.at[i]], rows.at[p], sem).wait()

            @pl.when(i + 1 < NCH)
            def _():
                pltpu.async_copy(h.at[idx_s.at[i + 1]], rows.at[pn], sem)

            pltpu.sync_copy(rows.at[p], sacc.at[idx_d.at[i]], add=True)
            return carry

        lax.fori_loop(0, NCH, body, 0)
        plsc.subcore_barrier()
        pltpu.sync_copy(
            sacc.at[pl.ds(s * SLAB, SLAB)],
            acc_out.at[pl.ds(c * NPAD + s * SLAB, SLAB)],
        )

    return _scatter


_scatter128 = _make_scatter(D_HID)
_scatter64 = _make_scatter(D_OUT)


def _k0_body(degp_ref, dinv_ref):
    d = degp_ref[0] + degp_ref[1] + 1.0
    dinv_ref[...] = lax.rsqrt(d)


def _k1_body(x_ref, dinv_ref, w_ref, o_ref):
    h = jnp.dot(x_ref[...], w_ref[...], preferred_element_type=jnp.float32)
    o_ref[...] = dinv_ref[...] * h


def _k2_body(acc_ref, h1p_ref, dinv_ref, b1_ref, w2_ref, o_ref):
    dinv = dinv_ref[...]
    t = dinv * (acc_ref[0] + acc_ref[1] + h1p_ref[...]) + b1_ref[...]
    r = jnp.maximum(t, 0.0)
    h2 = jnp.dot(r, w2_ref[...], preferred_element_type=jnp.float32)
    o_ref[...] = dinv * h2


def _k3_body(acc_ref, h2p_ref, dinv_ref, b2_ref, o_ref):
    t = dinv_ref[...] * (acc_ref[0] + acc_ref[1] + h2p_ref[...]) + b2_ref[...]
    m = jnp.max(t, axis=1, keepdims=True)
    e = jnp.exp(t - m)
    ssum = jnp.sum(e, axis=1, keepdims=True)
    o_ref[...] = (t - m) - jnp.log(ssum)


def _full(shape):
    return pl.BlockSpec(shape, lambda: tuple(0 for _ in shape))


def kernel(x, edge_index, W1, b1, W2, b2):
    ei = edge_index.astype(jnp.int32)
    src2d = ei[0].reshape(NT * NCH, CH)
    dst2d = ei[1].reshape(NT * NCH, CH)
    zeros1 = jnp.zeros((NPAD,), jnp.float32)
    zeros128 = jnp.zeros((NPAD, D_HID), jnp.float32)
    zeros64 = jnp.zeros((NPAD, D_OUT), jnp.float32)

    degflat = _deg_kernel(dst2d, zeros1)
    degp3 = degflat.reshape(NC, NPAD // 128, 128)

    dinv2d = pl.pallas_call(
        _k0_body,
        out_shape=jax.ShapeDtypeStruct((NPAD // 128, 128), jnp.float32),
    )(degp3)
    dinv = dinv2d.reshape(NPAD)[:N][:, None]

    h1p = pl.pallas_call(
        _k1_body,
        out_shape=jax.ShapeDtypeStruct((N, D_HID), jnp.float32),
    )(x, dinv, W1)

    acc1 = _scatter128(h1p, src2d, dst2d, zeros128).reshape(NC, NPAD, D_HID)

    h2p = pl.pallas_call(
        _k2_body,
        in_specs=[
            _full((NC, N, D_HID)),
            _full((N, D_HID)),
            _full((N, 1)),
            _full((1, D_HID)),
            _full((D_HID, D_OUT)),
        ],
        out_specs=_full((N, D_OUT)),
        out_shape=jax.ShapeDtypeStruct((N, D_OUT), jnp.float32),
    )(acc1, h1p, dinv, b1.reshape(1, D_HID), W2)

    acc2 = _scatter64(h2p, src2d, dst2d, zeros64).reshape(NC, NPAD, D_OUT)

    out = pl.pallas_call(
        _k3_body,
        in_specs=[
            _full((NC, N, D_OUT)),
            _full((N, D_OUT)),
            _full((N, 1)),
            _full((1, D_OUT)),
        ],
        out_specs=_full((N, D_OUT)),
        out_shape=jax.ShapeDtypeStruct((N, D_OUT), jnp.float32),
    )(acc2, h2p, dinv, b2.reshape(1, D_OUT))
    return out


# speedup vs baseline: 25.3017x; 25.3017x over previous
import functools

import jax
import jax.numpy as jnp
from jax import lax
from jax.experimental import pallas as pl
from jax.experimental.pallas import tpu as pltpu
from jax.experimental.pallas import tpu_sc as plsc

N = 10000
E = 320000
D_IN = 128
D_HID = 128
D_OUT = 64
DW = 128

NC = 2
NS = 16
NT = NC * NS
NPAD = 10240
SLAB = NPAD // NS
CH = 80
NCH = (E // NT) // CH

_MESH = plsc.VectorSubcoreMesh(
    core_axis_name="c", subcore_axis_name="s", num_cores=NC, num_subcores=NS
)


@functools.partial(
    pl.kernel,
    out_type=jax.ShapeDtypeStruct((NC * NPAD,), jnp.float32),
    mesh=_MESH,
    scratch_types=[
        pltpu.VMEM((NCH, CH), jnp.int32),
        pltpu.VMEM((CH,), jnp.float32),
        pltpu.VMEM_SHARED((NPAD,), jnp.float32),
    ],
)
def _deg_kernel(dst3d, zeros1, deg_out, idx_d, ones_v, sdeg):
    c = lax.axis_index("c")
    s = lax.axis_index("s")
    t = c * NS + s
    pltpu.sync_copy(zeros1.at[pl.ds(s * SLAB, SLAB)], sdeg.at[pl.ds(s * SLAB, SLAB)])
    pltpu.sync_copy(dst3d.at[t], idx_d)
    ones16 = jnp.full((16,), 1.0, dtype=jnp.float32)
    for j in range(CH // 16):
        ones_v[pl.ds(j * 16, 16)] = ones16
    plsc.subcore_barrier()

    def body(i, carry):
        pltpu.sync_copy(ones_v, sdeg.at[idx_d.at[i]], add=True)
        return carry

    lax.fori_loop(0, NCH, body, 0)
    plsc.subcore_barrier()
    pltpu.sync_copy(
        sdeg.at[pl.ds(s * SLAB, SLAB)],
        deg_out.at[pl.ds(c * NPAD + s * SLAB, SLAB)],
    )


@functools.partial(
    pl.kernel,
    out_type=jax.ShapeDtypeStruct((NC * NPAD, DW), jnp.float32),
    mesh=_MESH,
    scratch_types=[
        pltpu.VMEM((2, CH), jnp.int32),
        pltpu.VMEM((2, CH), jnp.int32),
        pltpu.VMEM((2, CH, DW), jnp.float32),
        pltpu.SemaphoreType.DMA,
        pltpu.SemaphoreType.DMA,
        pltpu.VMEM_SHARED((NPAD, DW), jnp.float32),
    ],
)
def _scatter_kernel(h, src3d, dst3d, zeros2, acc_out,
                    idx_s, idx_d, rows, gsem, isem, sacc):
    c = lax.axis_index("c")
    s = lax.axis_index("s")
    t = c * NS + s
    slab = pl.ds(s * SLAB, SLAB)
    pltpu.sync_copy(zeros2.at[slab], sacc.at[slab])
    pltpu.async_copy(src3d.at[t, 0], idx_s.at[0], isem)
    pltpu.async_copy(dst3d.at[t, 0], idx_d.at[0], isem)
    pltpu.async_copy(src3d.at[t, 1], idx_s.at[1], isem)
    pltpu.async_copy(dst3d.at[t, 1], idx_d.at[1], isem)
    pltpu.make_async_copy(src3d.at[t, 0], idx_s.at[0], isem).wait()
    pltpu.make_async_copy(dst3d.at[t, 0], idx_d.at[0], isem).wait()
    plsc.subcore_barrier()
    pltpu.async_copy(h.at[idx_s.at[0]], rows.at[0], gsem)

    def body(i, carry):
        pb = lax.rem(i, 2)
        pn = lax.rem(i + 1, 2)
        pltpu.make_async_copy(h.at[idx_s.at[pb]], rows.at[pb], gsem).wait()

        @pl.when(i + 1 < NCH)
        def _():
            pltpu.make_async_copy(src3d.at[t, i + 1], idx_s.at[pn], isem).wait()
            pltpu.make_async_copy(dst3d.at[t, i + 1], idx_d.at[pn], isem).wait()
            pltpu.async_copy(h.at[idx_s.at[pn]], rows.at[pn], gsem)

        pltpu.sync_copy(rows.at[pb], sacc.at[idx_d.at[pb]], add=True)

        @pl.when(i + 2 < NCH)
        def _():
            pltpu.async_copy(src3d.at[t, i + 2], idx_s.at[pb], isem)
            pltpu.async_copy(dst3d.at[t, i + 2], idx_d.at[pb], isem)

        return carry

    lax.fori_loop(0, NCH, body, 0)
    plsc.subcore_barrier()
    pltpu.sync_copy(sacc.at[slab], acc_out.at[pl.ds(c * NPAD + s * SLAB, SLAB)])


def _k0_body(degp_ref, dinv_ref):
    d = degp_ref[0] + degp_ref[1] + 1.0
    dinv_ref[...] = lax.rsqrt(d)


def _k1_body(x_ref, dinv_ref, w_ref, o_ref):
    h = jnp.dot(x_ref[...], w_ref[...], preferred_element_type=jnp.float32)
    o_ref[...] = dinv_ref[...] * h


def _k2_body(acc_ref, h1p_ref, dinv_ref, b1_ref, w2_ref, o_ref):
    dinv = dinv_ref[...]
    t = dinv * (acc_ref[0] + acc_ref[1] + h1p_ref[...]) + b1_ref[...]
    r = jnp.maximum(t, 0.0)
    h2 = jnp.dot(r, w2_ref[...], preferred_element_type=jnp.float32)
    h2p = dinv * h2
    o_ref[...] = jnp.concatenate([h2p, jnp.zeros_like(h2p)], axis=1)


def _k3_body(acc_ref, h2p_ref, dinv_ref, b2_ref, o_ref):
    acc = acc_ref[0] + acc_ref[1] + h2p_ref[...]
    t = dinv_ref[...] * acc[:, :D_OUT] + b2_ref[...]
    m = jnp.max(t, axis=1, keepdims=True)
    e = jnp.exp(t - m)
    ssum = jnp.sum(e, axis=1, keepdims=True)
    o_ref[...] = (t - m) - jnp.log(ssum)


def _full(shape):
    return pl.BlockSpec(shape, lambda i: tuple(0 for _ in shape))


def kernel(x, edge_index, W1, b1, W2, b2):
    ei = edge_index.astype(jnp.int32)
    src3d = ei[0].reshape(NT, NCH, CH)
    dst3d = ei[1].reshape(NT, NCH, CH)
    zeros1 = jnp.zeros((NPAD,), jnp.float32)
    zeros2 = jnp.zeros((NPAD, DW), jnp.float32)

    degflat = _deg_kernel(dst3d, zeros1)
    degp3 = degflat.reshape(NC, NPAD // 128, 128)

    dinv2d = pl.pallas_call(
        _k0_body,
        out_shape=jax.ShapeDtypeStruct((NPAD // 128, 128), jnp.float32),
    )(degp3)
    dinv = dinv2d.reshape(NPAD)[:N][:, None]

    h1p = pl.pallas_call(
        _k1_body,
        out_shape=jax.ShapeDtypeStruct((N, D_HID), jnp.float32),
    )(x, dinv, W1)

    acc1 = _scatter_kernel(h1p, src3d, dst3d, zeros2).reshape(NC, NPAD, DW)

    h2p = pl.pallas_call(
        _k2_body,
        grid=(1,),
        in_specs=[
            _full((NC, N, DW)),
            _full((N, D_HID)),
            _full((N, 1)),
            _full((1, D_HID)),
            _full((D_HID, D_OUT)),
        ],
        out_specs=_full((N, DW)),
        out_shape=jax.ShapeDtypeStruct((N, DW), jnp.float32),
    )(acc1, h1p, dinv, b1.reshape(1, D_HID), W2)

    acc2 = _scatter_kernel(h2p, src3d, dst3d, zeros2).reshape(NC, NPAD, DW)

    out = pl.pallas_call(
        _k3_body,
        grid=(1,),
        in_specs=[
            _full((NC, N, DW)),
            _full((N, DW)),
            _full((N, 1)),
            _full((1, D_OUT)),
        ],
        out_specs=_full((N, D_OUT)),
        out_shape=jax.ShapeDtypeStruct((N, D_OUT), jnp.float32),
    )(acc2, h2p, dinv, b2.reshape(1, D_OUT))
    return out


# speedup vs baseline: 34.2600x; 1.3541x over previous
import functools

import jax
import jax.numpy as jnp
from jax import lax
from jax.experimental import pallas as pl
from jax.experimental.pallas import tpu as pltpu
from jax.experimental.pallas import tpu_sc as plsc

N = 10000
E = 320000
D_IN = 128
D_HID = 128
D_OUT = 64
DW = 128

NC = 2
NS = 16
NT = NC * NS
NPAD = 10240
SLAB = NPAD // NS
CH = 80
NCH = (E // NT) // CH

_MESH = plsc.VectorSubcoreMesh(
    core_axis_name="c", subcore_axis_name="s", num_cores=NC, num_subcores=NS
)


@functools.partial(
    pl.kernel,
    out_type=jax.ShapeDtypeStruct((NC * NPAD,), jnp.float32),
    mesh=_MESH,
    scratch_types=[
        pltpu.VMEM((NCH, CH), jnp.int32),
        pltpu.VMEM((CH,), jnp.float32),
        pltpu.VMEM_SHARED((NPAD,), jnp.float32),
    ],
)
def _deg_kernel(dst3d, zeros1, deg_out, idx_d, ones_v, sdeg):
    c = lax.axis_index("c")
    s = lax.axis_index("s")
    t = c * NS + s
    pltpu.sync_copy(zeros1.at[pl.ds(s * SLAB, SLAB)], sdeg.at[pl.ds(s * SLAB, SLAB)])
    pltpu.sync_copy(dst3d.at[t], idx_d)
    ones16 = jnp.full((16,), 1.0, dtype=jnp.float32)
    for j in range(CH // 16):
        ones_v[pl.ds(j * 16, 16)] = ones16
    plsc.subcore_barrier()

    def body(i, carry):
        pltpu.sync_copy(ones_v, sdeg.at[idx_d.at[i]], add=True)
        return carry

    lax.fori_loop(0, NCH, body, 0)
    plsc.subcore_barrier()
    pltpu.sync_copy(
        sdeg.at[pl.ds(s * SLAB, SLAB)],
        deg_out.at[pl.ds(c * NPAD + s * SLAB, SLAB)],
    )


NI = 6
NR = 4


@functools.partial(
    pl.kernel,
    out_type=jax.ShapeDtypeStruct((NC * NPAD, DW), jnp.float32),
    mesh=_MESH,
    scratch_types=[
        pltpu.VMEM((NI, CH), jnp.int32),
        pltpu.VMEM((NI, CH), jnp.int32),
        pltpu.VMEM((NR, CH, DW), jnp.float32),
        pltpu.SemaphoreType.DMA,
        pltpu.SemaphoreType.DMA,
        pltpu.SemaphoreType.DMA,
        pltpu.VMEM_SHARED((NPAD, DW), jnp.float32),
    ],
)
def _scatter_kernel(h, src3d, dst3d, zeros2, acc_out,
                    idx_s, idx_d, rows, gsem, isem, ssem, sacc):
    c = lax.axis_index("c")
    s = lax.axis_index("s")
    t = c * NS + s
    slab = pl.ds(s * SLAB, SLAB)
    pltpu.sync_copy(zeros2.at[slab], sacc.at[slab])
    plsc.subcore_barrier()

    def load_idx(j, slot):
        pltpu.async_copy(src3d.at[t, j], idx_s.at[slot], isem)
        pltpu.async_copy(dst3d.at[t, j], idx_d.at[slot], isem)

    def wait_idx(slot):
        pltpu.make_async_copy(src3d.at[t, 0], idx_s.at[slot], isem).wait()
        pltpu.make_async_copy(dst3d.at[t, 0], idx_d.at[slot], isem).wait()

    def gather(slot_i, slot_r):
        pltpu.async_copy(h.at[idx_s.at[slot_i]], rows.at[slot_r], gsem)

    def wait_gather(slot_i, slot_r):
        pltpu.make_async_copy(h.at[idx_s.at[slot_i]], rows.at[slot_r], gsem).wait()

    def scat(slot_i, slot_r):
        pltpu.async_copy(rows.at[slot_r], sacc.at[idx_d.at[slot_i]], ssem, add=True)

    def wait_scat(slot_i, slot_r):
        pltpu.make_async_copy(
            rows.at[slot_r], sacc.at[idx_d.at[slot_i]], ssem
        ).wait()

    for j in range(4):
        load_idx(j, j)
    wait_idx(0)
    gather(0, 0)
    wait_idx(1)
    gather(1, 1)

    def body(i, carry):
        ri = lax.rem(i, NR)
        ii = lax.rem(i, NI)
        wait_gather(ii, ri)
        scat(ii, ri)

        @pl.when(i - 2 >= 0)
        def _():
            wait_scat(lax.rem(i - 2, NI), lax.rem(i - 2, NR))

        @pl.when(i + 2 < NCH)
        def _():
            wait_idx(lax.rem(i + 2, NI))
            gather(lax.rem(i + 2, NI), lax.rem(i + 2, NR))

        @pl.when(i + 4 < NCH)
        def _():
            load_idx(i + 4, lax.rem(i + 4, NI))

        return carry

    lax.fori_loop(0, NCH, body, 0)
    wait_scat(lax.rem(NCH - 2, NI), lax.rem(NCH - 2, NR))
    wait_scat(lax.rem(NCH - 1, NI), lax.rem(NCH - 1, NR))
    plsc.subcore_barrier()
    pltpu.sync_copy(sacc.at[slab], acc_out.at[pl.ds(c * NPAD + s * SLAB, SLAB)])


def _k0_body(degp_ref, dinv_ref):
    d = degp_ref[0] + degp_ref[1] + 1.0
    dinv_ref[...] = lax.rsqrt(d)


def _k1_body(x_ref, dinv_ref, w_ref, o_ref):
    h = jnp.dot(x_ref[...], w_ref[...], preferred_element_type=jnp.float32)
    o_ref[...] = dinv_ref[...] * h


def _k2_body(acc_ref, h1p_ref, dinv_ref, b1_ref, w2_ref, o_ref):
    dinv = dinv_ref[...]
    t = dinv * (acc_ref[0] + acc_ref[1] + h1p_ref[...]) + b1_ref[...]
    r = jnp.maximum(t, 0.0)
    h2 = jnp.dot(r, w2_ref[...], preferred_element_type=jnp.float32)
    h2p = dinv * h2
    o_ref[...] = jnp.concatenate([h2p, jnp.zeros_like(h2p)], axis=1)


def _k3_body(acc_ref, h2p_ref, dinv_ref, b2_ref, o_ref):
    acc = acc_ref[0] + acc_ref[1] + h2p_ref[...]
    t = dinv_ref[...] * acc[:, :D_OUT] + b2_ref[...]
    m = jnp.max(t, axis=1, keepdims=True)
    e = jnp.exp(t - m)
    ssum = jnp.sum(e, axis=1, keepdims=True)
    o_ref[...] = (t - m) - jnp.log(ssum)


def _full(shape):
    return pl.BlockSpec(shape, lambda i: tuple(0 for _ in shape))


def kernel(x, edge_index, W1, b1, W2, b2):
    ei = edge_index.astype(jnp.int32)
    src3d = ei[0].reshape(NT, NCH, CH)
    dst3d = ei[1].reshape(NT, NCH, CH)
    zeros1 = jnp.zeros((NPAD,), jnp.float32)
    zeros2 = jnp.zeros((NPAD, DW), jnp.float32)

    degflat = _deg_kernel(dst3d, zeros1)
    degp3 = degflat.reshape(NC, NPAD // 128, 128)

    dinv2d = pl.pallas_call(
        _k0_body,
        out_shape=jax.ShapeDtypeStruct((NPAD // 128, 128), jnp.float32),
    )(degp3)
    dinv = dinv2d.reshape(NPAD)[:N][:, None]

    h1p = pl.pallas_call(
        _k1_body,
        out_shape=jax.ShapeDtypeStruct((N, D_HID), jnp.float32),
    )(x, dinv, W1)

    acc1 = _scatter_kernel(h1p, src3d, dst3d, zeros2).reshape(NC, NPAD, DW)

    h2p = pl.pallas_call(
        _k2_body,
        grid=(1,),
        in_specs=[
            _full((NC, N, DW)),
            _full((N, D_HID)),
            _full((N, 1)),
            _full((1, D_HID)),
            _full((D_HID, D_OUT)),
        ],
        out_specs=_full((N, DW)),
        out_shape=jax.ShapeDtypeStruct((N, DW), jnp.float32),
    )(acc1, h1p, dinv, b1.reshape(1, D_HID), W2)

    acc2 = _scatter_kernel(h2p, src3d, dst3d, zeros2).reshape(NC, NPAD, DW)

    out = pl.pallas_call(
        _k3_body,
        grid=(1,),
        in_specs=[
            _full((NC, N, DW)),
            _full((N, DW)),
            _full((N, 1)),
            _full((1, D_OUT)),
        ],
        out_specs=_full((N, D_OUT)),
        out_shape=jax.ShapeDtypeStruct((N, D_OUT), jnp.float32),
    )(acc2, h2p, dinv, b2.reshape(1, D_OUT))
    return out


# speedup vs baseline: 39.0754x; 1.1406x over previous
import functools

import jax
import jax.numpy as jnp
from jax import lax
from jax.experimental import pallas as pl
from jax.experimental.pallas import tpu as pltpu
from jax.experimental.pallas import tpu_sc as plsc

N = 10000
E = 320000
D_IN = 128
D_HID = 128
D_OUT = 64
DW = 128

NC = 2
NS = 16
NT = NC * NS
NPAD = 10240
SLAB = NPAD // NS
CH = 80
NCH = (E // NT) // CH

_MESH = plsc.VectorSubcoreMesh(
    core_axis_name="c", subcore_axis_name="s", num_cores=NC, num_subcores=NS
)


@functools.partial(
    pl.kernel,
    out_type=jax.ShapeDtypeStruct((NC * NPAD,), jnp.float32),
    mesh=_MESH,
    scratch_types=[
        pltpu.VMEM((NCH, CH), jnp.int32),
        pltpu.VMEM((CH,), jnp.float32),
        pltpu.VMEM_SHARED((NPAD,), jnp.float32),
    ],
)
def _deg_kernel(dst3d, zeros1, deg_out, idx_d, ones_v, sdeg):
    c = lax.axis_index("c")
    s = lax.axis_index("s")
    t = c * NS + s
    pltpu.sync_copy(zeros1.at[pl.ds(s * SLAB, SLAB)], sdeg.at[pl.ds(s * SLAB, SLAB)])
    pltpu.sync_copy(dst3d.at[t], idx_d)
    ones16 = jnp.full((16,), 1.0, dtype=jnp.float32)
    for j in range(CH // 16):
        ones_v[pl.ds(j * 16, 16)] = ones16
    plsc.subcore_barrier()

    def body(i, carry):
        pltpu.sync_copy(ones_v, sdeg.at[idx_d.at[i]], add=True)
        return carry

    lax.fori_loop(0, NCH, body, 0)
    plsc.subcore_barrier()
    pltpu.sync_copy(
        sdeg.at[pl.ds(s * SLAB, SLAB)],
        deg_out.at[pl.ds(c * NPAD + s * SLAB, SLAB)],
    )


def _make_scatter(ds, nr, ni, slag, gla, ila, tc_tiling=True):
    assert ni >= ila + slag + 1 and nr >= gla + slag

    @functools.partial(
        pl.kernel,
        out_type=jax.ShapeDtypeStruct((NC * NPAD, ds), jnp.float32),
        mesh=_MESH,
        compiler_params=pltpu.CompilerParams(use_tc_tiling_on_sc=tc_tiling),
        scratch_types=[
            pltpu.VMEM((ni, CH), jnp.int32),
            pltpu.VMEM((ni, CH), jnp.int32),
            pltpu.VMEM((nr, CH, ds), jnp.float32),
            pltpu.SemaphoreType.DMA,
            pltpu.SemaphoreType.DMA,
            pltpu.SemaphoreType.DMA,
            pltpu.VMEM_SHARED((NPAD, ds), jnp.float32),
        ],
    )
    def _scatter(h, src3d, dst3d, zeros2, acc_out,
                 idx_s, idx_d, rows, gsem, isem, ssem, sacc):
        c = lax.axis_index("c")
        s = lax.axis_index("s")
        t = c * NS + s
        slab = pl.ds(s * SLAB, SLAB)
        pltpu.sync_copy(zeros2.at[slab], sacc.at[slab])
        plsc.subcore_barrier()

        def load_idx(j, slot):
            pltpu.async_copy(src3d.at[t, j], idx_s.at[slot], isem)
            pltpu.async_copy(dst3d.at[t, j], idx_d.at[slot], isem)

        def wait_idx(slot):
            pltpu.make_async_copy(src3d.at[t, 0], idx_s.at[slot], isem).wait()
            pltpu.make_async_copy(dst3d.at[t, 0], idx_d.at[slot], isem).wait()

        def gather(slot_i, slot_r):
            pltpu.async_copy(h.at[idx_s.at[slot_i]], rows.at[slot_r], gsem)

        def wait_gather(slot_i, slot_r):
            pltpu.make_async_copy(
                h.at[idx_s.at[slot_i]], rows.at[slot_r], gsem
            ).wait()

        def scat(slot_i, slot_r):
            pltpu.async_copy(
                rows.at[slot_r], sacc.at[idx_d.at[slot_i]], ssem, add=True
            )

        def wait_scat(slot_i, slot_r):
            pltpu.make_async_copy(
                rows.at[slot_r], sacc.at[idx_d.at[slot_i]], ssem
            ).wait()

        for j in range(ila):
            load_idx(j, j)
        for j in range(gla):
            wait_idx(j)
            gather(j, j)

        def body(i, carry):
            ri = lax.rem(i, nr)
            ii = lax.rem(i, ni)
            wait_gather(ii, ri)
            scat(ii, ri)

            @pl.when(i - slag >= 0)
            def _():
                wait_scat(lax.rem(i - slag, ni), lax.rem(i - slag, nr))

            @pl.when(i + gla < NCH)
            def _():
                wait_idx(lax.rem(i + gla, ni))
                gather(lax.rem(i + gla, ni), lax.rem(i + gla, nr))

            @pl.when(i + ila < NCH)
            def _():
                load_idx(i + ila, lax.rem(i + ila, ni))

            return carry

        lax.fori_loop(0, NCH, body, 0)
        for j in range(slag, 0, -1):
            wait_scat(lax.rem(NCH - j, ni), lax.rem(NCH - j, nr))
        plsc.subcore_barrier()
        pltpu.sync_copy(
            sacc.at[slab], acc_out.at[pl.ds(c * NPAD + s * SLAB, SLAB)]
        )

    return _scatter


_scatter_l1 = _make_scatter(ds=DW, nr=4, ni=6, slag=2, gla=2, ila=3)
_scatter_l2 = _make_scatter(ds=D_OUT, nr=6, ni=10, slag=3, gla=3, ila=6,
                            tc_tiling=False)


def _k0_body(degp_ref, dinv_ref):
    d = degp_ref[0] + degp_ref[1] + 1.0
    dinv_ref[...] = lax.rsqrt(d)


def _k1_body(x_ref, dinv_ref, w_ref, o_ref):
    h = jnp.dot(x_ref[...], w_ref[...], preferred_element_type=jnp.float32)
    o_ref[...] = dinv_ref[...] * h


def _k2_body(acc_ref, h1p_ref, dinv_ref, b1_ref, w2_ref, o_ref):
    dinv = dinv_ref[...]
    t = dinv * (acc_ref[0] + acc_ref[1] + h1p_ref[...]) + b1_ref[...]
    r = jnp.maximum(t, 0.0)
    h2 = jnp.dot(r, w2_ref[...], preferred_element_type=jnp.float32)
    o_ref[...] = dinv * h2


def _k3_body(acc_ref, h2p_ref, dinv_ref, b2_ref, o_ref):
    acc = acc_ref[0] + acc_ref[1] + h2p_ref[...]
    t = dinv_ref[...] * acc + b2_ref[...]
    m = jnp.max(t, axis=1, keepdims=True)
    e = jnp.exp(t - m)
    ssum = jnp.sum(e, axis=1, keepdims=True)
    o_ref[...] = (t - m) - jnp.log(ssum)


def _full(shape):
    return pl.BlockSpec(shape, lambda i: tuple(0 for _ in shape))


def kernel(x, edge_index, W1, b1, W2, b2):
    ei = edge_index.astype(jnp.int32)
    src3d = ei[0].reshape(NT, NCH, CH)
    dst3d = ei[1].reshape(NT, NCH, CH)
    zeros1 = jnp.zeros((NPAD,), jnp.float32)
    zeros2 = jnp.zeros((NPAD, DW), jnp.float32)
    zeros64 = jnp.zeros((NPAD, D_OUT), jnp.float32)

    degflat = _deg_kernel(dst3d, zeros1)
    degp3 = degflat.reshape(NC, NPAD // 128, 128)

    dinv2d = pl.pallas_call(
        _k0_body,
        out_shape=jax.ShapeDtypeStruct((NPAD // 128, 128), jnp.float32),
    )(degp3)
    dinv = dinv2d.reshape(NPAD)[:N][:, None]

    h1p = pl.pallas_call(
        _k1_body,
        out_shape=jax.ShapeDtypeStruct((N, D_HID), jnp.float32),
    )(x, dinv, W1)

    acc1 = _scatter_l1(h1p, src3d, dst3d, zeros2).reshape(NC, NPAD, DW)

    h2p = pl.pallas_call(
        _k2_body,
        grid=(1,),
        in_specs=[
            _full((NC, N, DW)),
            _full((N, D_HID)),
            _full((N, 1)),
            _full((1, D_HID)),
            _full((D_HID, D_OUT)),
        ],
        out_specs=_full((N, D_OUT)),
        out_shape=jax.ShapeDtypeStruct((N, D_OUT), jnp.float32),
    )(acc1, h1p, dinv, b1.reshape(1, D_HID), W2)

    acc2 = _scatter_l2(h2p, src3d, dst3d, zeros64).reshape(NC, NPAD, D_OUT)

    out = pl.pallas_call(
        _k3_body,
        grid=(1,),
        in_specs=[
            _full((NC, N, D_OUT)),
            _full((N, D_OUT)),
            _full((N, 1)),
            _full((1, D_OUT)),
        ],
        out_specs=_full((N, D_OUT)),
        out_shape=jax.ShapeDtypeStruct((N, D_OUT), jnp.float32),
    )(acc2, h2p, dinv, b2.reshape(1, D_OUT))
    return out


# speedup vs baseline: 39.1884x; 1.0029x over previous
import functools

import jax
import jax.numpy as jnp
from jax import lax
from jax.experimental import pallas as pl
from jax.experimental.pallas import tpu as pltpu
from jax.experimental.pallas import tpu_sc as plsc

N = 10000
E = 320000
D_IN = 128
D_HID = 128
D_OUT = 64
DW = 128

NC = 2
NS = 16
NT = NC * NS
NPAD = 10240
SLAB = NPAD // NS
CH = 80
NCH = (E // NT) // CH

_MESH = plsc.VectorSubcoreMesh(
    core_axis_name="c", subcore_axis_name="s", num_cores=NC, num_subcores=NS
)


@functools.partial(
    pl.kernel,
    out_type=jax.ShapeDtypeStruct((NC * NPAD,), jnp.float32),
    mesh=_MESH,
    scratch_types=[
        pltpu.VMEM((NCH, CH), jnp.int32),
        pltpu.VMEM((CH,), jnp.float32),
        pltpu.VMEM_SHARED((NPAD,), jnp.float32),
    ],
)
def _deg_kernel(dst3d, zeros1, deg_out, idx_d, ones_v, sdeg):
    c = lax.axis_index("c")
    s = lax.axis_index("s")
    t = c * NS + s
    pltpu.sync_copy(zeros1.at[pl.ds(s * SLAB, SLAB)], sdeg.at[pl.ds(s * SLAB, SLAB)])
    pltpu.sync_copy(dst3d.at[t], idx_d)
    ones16 = jnp.full((16,), 1.0, dtype=jnp.float32)
    for j in range(CH // 16):
        ones_v[pl.ds(j * 16, 16)] = ones16
    plsc.subcore_barrier()

    def body(i, carry):
        pltpu.sync_copy(ones_v, sdeg.at[idx_d.at[i]], add=True)
        return carry

    lax.fori_loop(0, NCH, body, 0)
    plsc.subcore_barrier()
    pltpu.sync_copy(
        sdeg.at[pl.ds(s * SLAB, SLAB)],
        deg_out.at[pl.ds(c * NPAD + s * SLAB, SLAB)],
    )


def _make_scatter(ds, nr, ni, slag, gla, ila, tc_tiling=True):
    assert ni >= ila + slag + 1 and nr >= gla + slag

    @functools.partial(
        pl.kernel,
        out_type=jax.ShapeDtypeStruct((NC * NPAD, ds), jnp.float32),
        mesh=_MESH,
        compiler_params=pltpu.CompilerParams(use_tc_tiling_on_sc=tc_tiling),
        scratch_types=[
            pltpu.VMEM((ni, CH), jnp.int32),
            pltpu.VMEM((ni, CH), jnp.int32),
            pltpu.VMEM((nr, CH, ds), jnp.float32),
            pltpu.SemaphoreType.DMA,
            pltpu.SemaphoreType.DMA,
            pltpu.SemaphoreType.DMA,
            pltpu.VMEM_SHARED((NPAD, ds), jnp.float32),
        ],
    )
    def _scatter(h, src3d, dst3d, zeros2, acc_out,
                 idx_s, idx_d, rows, gsem, isem, ssem, sacc):
        c = lax.axis_index("c")
        s = lax.axis_index("s")
        t = c * NS + s
        slab = pl.ds(s * SLAB, SLAB)
        pltpu.sync_copy(zeros2.at[slab], sacc.at[slab])
        plsc.subcore_barrier()

        def load_idx(j, slot):
            pltpu.async_copy(src3d.at[t, j], idx_s.at[slot], isem)
            pltpu.async_copy(dst3d.at[t, j], idx_d.at[slot], isem)

        def wait_idx(slot):
            pltpu.make_async_copy(src3d.at[t, 0], idx_s.at[slot], isem).wait()
            pltpu.make_async_copy(dst3d.at[t, 0], idx_d.at[slot], isem).wait()

        def gather(slot_i, slot_r):
            pltpu.async_copy(h.at[idx_s.at[slot_i]], rows.at[slot_r], gsem)

        def wait_gather(slot_i, slot_r):
            pltpu.make_async_copy(
                h.at[idx_s.at[slot_i]], rows.at[slot_r], gsem
            ).wait()

        def scat(slot_i, slot_r):
            pltpu.async_copy(
                rows.at[slot_r], sacc.at[idx_d.at[slot_i]], ssem, add=True
            )

        def wait_scat(slot_i, slot_r):
            pltpu.make_async_copy(
                rows.at[slot_r], sacc.at[idx_d.at[slot_i]], ssem
            ).wait()

        for j in range(ila):
            load_idx(j, j)
        for j in range(gla):
            wait_idx(j)
            gather(j, j)

        def body(i, carry):
            ri = lax.rem(i, nr)
            ii = lax.rem(i, ni)
            wait_gather(ii, ri)
            scat(ii, ri)

            @pl.when(i - slag >= 0)
            def _():
                wait_scat(lax.rem(i - slag, ni), lax.rem(i - slag, nr))

            @pl.when(i + gla < NCH)
            def _():
                wait_idx(lax.rem(i + gla, ni))
                gather(lax.rem(i + gla, ni), lax.rem(i + gla, nr))

            @pl.when(i + ila < NCH)
            def _():
                load_idx(i + ila, lax.rem(i + ila, ni))

            return carry

        lax.fori_loop(0, NCH, body, 0)
        for j in range(slag, 0, -1):
            wait_scat(lax.rem(NCH - j, ni), lax.rem(NCH - j, nr))
        plsc.subcore_barrier()
        pltpu.sync_copy(
            sacc.at[slab], acc_out.at[pl.ds(c * NPAD + s * SLAB, SLAB)]
        )

    return _scatter


_scatter_l1 = _make_scatter(ds=DW, nr=4, ni=6, slag=2, gla=2, ila=3)
_scatter_l2 = _make_scatter(ds=D_OUT, nr=6, ni=10, slag=3, gla=3, ila=6,
                            tc_tiling=False)


def _k1_body(x_ref, deg2_ref, w_ref, o_ref, dinv_ref):
    d2 = deg2_ref[...]
    d = d2[:, 0:1] + d2[:, 1:2] + 1.0
    dinv = lax.rsqrt(d)
    dinv_ref[...] = dinv
    h = jnp.dot(x_ref[...], w_ref[...], preferred_element_type=jnp.float32)
    o_ref[...] = dinv * h


def _k2_body(acc_ref, h1p_ref, dinv_ref, b1_ref, w2_ref, o_ref):
    dinv = dinv_ref[...]
    t = dinv * (acc_ref[0] + acc_ref[1] + h1p_ref[...]) + b1_ref[...]
    r = jnp.maximum(t, 0.0)
    h2 = jnp.dot(r, w2_ref[...], preferred_element_type=jnp.float32)
    o_ref[...] = dinv * h2


def _k3_body(acc_ref, h2p_ref, dinv_ref, b2_ref, o_ref):
    acc = acc_ref[0] + acc_ref[1] + h2p_ref[...]
    t = dinv_ref[...] * acc + b2_ref[...]
    m = jnp.max(t, axis=1, keepdims=True)
    e = jnp.exp(t - m)
    ssum = jnp.sum(e, axis=1, keepdims=True)
    o_ref[...] = (t - m) - jnp.log(ssum)


def _full(shape):
    return pl.BlockSpec(shape, lambda i: tuple(0 for _ in shape))


def kernel(x, edge_index, W1, b1, W2, b2):
    ei = edge_index.astype(jnp.int32)
    src3d = ei[0].reshape(NT, NCH, CH)
    dst3d = ei[1].reshape(NT, NCH, CH)
    zeros1 = jnp.zeros((NPAD,), jnp.float32)
    zeros2 = jnp.zeros((NPAD, DW), jnp.float32)
    zeros64 = jnp.zeros((NPAD, D_OUT), jnp.float32)

    degflat = _deg_kernel(dst3d, zeros1)
    deg2 = degflat.reshape(NC, NPAD).T[:N]

    h1p, dinv = pl.pallas_call(
        _k1_body,
        out_shape=[
            jax.ShapeDtypeStruct((N, D_HID), jnp.float32),
            jax.ShapeDtypeStruct((N, 1), jnp.float32),
        ],
    )(x, deg2, W1)

    acc1 = _scatter_l1(h1p, src3d, dst3d, zeros2).reshape(NC, NPAD, DW)

    h2p = pl.pallas_call(
        _k2_body,
        grid=(1,),
        in_specs=[
            _full((NC, N, DW)),
            _full((N, D_HID)),
            _full((N, 1)),
            _full((1, D_HID)),
            _full((D_HID, D_OUT)),
        ],
        out_specs=_full((N, D_OUT)),
        out_shape=jax.ShapeDtypeStruct((N, D_OUT), jnp.float32),
    )(acc1, h1p, dinv, b1.reshape(1, D_HID), W2)

    acc2 = _scatter_l2(h2p, src3d, dst3d, zeros64).reshape(NC, NPAD, D_OUT)

    out = pl.pallas_call(
        _k3_body,
        grid=(1,),
        in_specs=[
            _full((NC, N, D_OUT)),
            _full((N, D_OUT)),
            _full((N, 1)),
            _full((1, D_OUT)),
        ],
        out_specs=_full((N, D_OUT)),
        out_shape=jax.ShapeDtypeStruct((N, D_OUT), jnp.float32),
    )(acc2, h2p, dinv, b2.reshape(1, D_OUT))
    return out
